# baseline (speedup 1.0000x reference)
import jax
import jax.numpy as jnp
from jax.experimental import pallas as pl
from jax.experimental.pallas import tpu as pltpu

_K = 5
_D = 512
_N = 8192
_BQ = 256
_NEG = float("-inf")
_BIGI = 2**30


def _topk_body(q_ref, e_ref, s_ref, i_ref):
    vals = jax.lax.dot_general(
        q_ref[...], e_ref[...], (((1,), (1,)), ((), ())),
        preferred_element_type=jnp.float32)

    iota = jax.lax.broadcasted_iota(jnp.int32, vals.shape, 1)

    ss, ii = [], []
    for t in range(_K):
        m = jnp.max(vals, axis=1, keepdims=True)
        hit = vals == m
        sel = jnp.min(jnp.where(hit, iota, _BIGI), axis=1, keepdims=True)
        ss.append(m)
        ii.append(sel)
        if t + 1 < _K:
            vals = jnp.where(iota == sel, _NEG, vals)

    s_ref[...] = jnp.concatenate(ss, axis=1)
    i_ref[...] = jnp.concatenate(ii, axis=1)


def kernel(query, embeddings, top_k):
    del top_k
    b, s, d = query.shape
    q2 = query.reshape(b * s, d)

    norm = jnp.linalg.norm(embeddings, axis=-1, keepdims=True)
    en = embeddings / jnp.maximum(norm, 1e-12)
    qnorm = jnp.linalg.norm(q2, axis=-1, keepdims=True)
    q2 = q2 / jnp.maximum(qnorm, 1e-12)

    nq = b * s
    scores, indices = pl.pallas_call(
        _topk_body,
        grid=(nq // _BQ,),
        in_specs=[
            pl.BlockSpec((_BQ, _D), lambda i: (i, 0)),
            pl.BlockSpec((_N, _D), lambda i: (0, 0)),
        ],
        out_specs=[
            pl.BlockSpec((_BQ, _K), lambda i: (i, 0)),
            pl.BlockSpec((_BQ, _K), lambda i: (i, 0)),
        ],
        out_shape=[
            jax.ShapeDtypeStruct((nq, _K), jnp.float32),
            jax.ShapeDtypeStruct((nq, _K), jnp.int32),
        ],
        compiler_params=pltpu.CompilerParams(
            dimension_semantics=("parallel",)),
    )(q2, en)

    return scores.reshape(b, s, _K), indices.reshape(b, s, _K)
